# baseline (speedup 1.0000x reference)
import functools

import jax
import jax.numpy as jnp
from jax import lax
from jax.experimental import pallas as pl
from jax.experimental.pallas import tpu as pltpu
from jax.experimental.pallas import tpu_sc as plsc

_BLK = 128
_K = 8
_FP = 28
_BM = 1024


def _make_gather(NB, D):
    info = plsc.get_sparse_core_info()
    NC, NS = info.num_cores, info.num_subcores
    NW = NC * NS
    nb_w = NB // NW
    n_grp = nb_w // _K
    mesh = plsc.VectorSubcoreMesh(core_axis_name="c", subcore_axis_name="s")

    @functools.partial(
        pl.kernel,
        mesh=mesh,
        out_type=jax.ShapeDtypeStruct((NB, _BLK, D), jnp.float32),
        scratch_types=[
            pltpu.VMEM((_K, _BLK), jnp.int32),
            pltpu.VMEM((_K, _BLK, D), jnp.float32),
            pltpu.SemaphoreType.DMA,
        ],
        compiler_params=pltpu.CompilerParams(use_tc_tiling_on_sc=False),
    )
    def gather(tab_hbm, idx_hbm, out_hbm, idx_v, rows_v, sem):
        wid = lax.axis_index("s") * NC + lax.axis_index("c")
        base = wid * nb_w

        def group(g, carry):
            blk0 = base + g * _K
            pltpu.sync_copy(idx_hbm.at[pl.ds(blk0, _K)], idx_v)
            copies = [
                pltpu.async_copy(tab_hbm.at[idx_v.at[j]], rows_v.at[j], sem)
                for j in range(_K)
            ]
            for cp in copies:
                cp.wait()
            pltpu.sync_copy(rows_v, out_hbm.at[pl.ds(blk0, _K)])
            return carry

        lax.fori_loop(0, n_grp, group, 0)

    return gather


def _mlp(emb4, num, W1e, W1n, b1, W2, b2, W3, b3, W4, b4):
    G = emb4.shape[0]
    T = emb4.shape[1]
    Bt = G * 8

    def body(emb_ref, num_ref, w1e_ref, w1n_ref, b1_ref, w2_ref, b2_ref,
             w3_ref, b3_ref, w4_ref, b4_ref, out_ref):
        h = jnp.dot(num_ref[...], w1n_ref[...], preferred_element_type=jnp.float32)
        for t in range(T):
            xt = emb_ref[:, t].reshape(_BM, 128)
            h = h + jnp.dot(xt, w1e_ref[t], preferred_element_type=jnp.float32)
        h = jnp.maximum(h + b1_ref[...], 0.0)
        h = jnp.maximum(jnp.dot(h, w2_ref[...], preferred_element_type=jnp.float32) + b2_ref[...], 0.0)
        h = jnp.maximum(jnp.dot(h, w3_ref[...], preferred_element_type=jnp.float32) + b3_ref[...], 0.0)
        out_ref[...] = jnp.dot(h, w4_ref[...], preferred_element_type=jnp.float32) + b4_ref[...]

    def full(a):
        nd = a.ndim
        return pl.BlockSpec(a.shape, lambda i, _nd=nd: (0,) * _nd)

    return pl.pallas_call(
        body,
        grid=(Bt // _BM,),
        in_specs=[
            pl.BlockSpec((_BM // 8, T, 8, 128), lambda i: (i, 0, 0, 0)),
            pl.BlockSpec((_BM, num.shape[1]), lambda i: (i, 0)),
            full(W1e), full(W1n), full(b1),
            full(W2), full(b2), full(W3), full(b3), full(W4), full(b4),
        ],
        out_specs=pl.BlockSpec((_BM, 1), lambda i: (i, 0)),
        out_shape=jax.ShapeDtypeStruct((Bt, 1), jnp.float32),
    )(emb4, num, W1e, W1n, b1, W2, b2, W3, b3, W4, b4)


def kernel(categorical_inputs, numeric_inputs, tables, W1, b1, W2, b2, W3, b3, W4, b4):
    B, F = categorical_inputs.shape
    _, V, D = tables.shape
    T = _FP * D // 128
    tab_flat = tables.reshape(F * V, D)

    flat = categorical_inputs + jnp.arange(F, dtype=jnp.int32) * V
    pad = (jnp.arange(B, dtype=jnp.int32) * (_FP - F))[:, None] + jnp.arange(
        _FP - F, dtype=jnp.int32)
    flat = jnp.concatenate([flat, pad], axis=1)
    flat = flat.reshape(B // 8, 8, T, 4).transpose(0, 2, 1, 3)
    NB = (B * _FP) // _BLK
    idx_blocked = flat.reshape(NB, _BLK)

    emb = _make_gather(NB, D)(tab_flat, idx_blocked)
    return emb.reshape(-1)[:B]
    emb4 = emb.reshape(B // 8, T, 8, 128)

    W1e = jnp.pad(W1[: F * D], ((0, _FP * D - F * D), (0, 0))).reshape(T, 128, -1)
    W1n = W1[F * D:]
    out = _mlp(
        emb4, numeric_inputs,
        W1e, W1n, b1.reshape(1, -1),
        W2, b2.reshape(1, -1), W3, b3.reshape(1, -1), W4, b4.reshape(1, -1),
    )
    return out.reshape(B)


# speedup vs baseline: 3.7809x; 3.7809x over previous
import functools

import jax
import jax.numpy as jnp
from jax import lax
from jax.experimental import pallas as pl
from jax.experimental.pallas import tpu as pltpu
from jax.experimental.pallas import tpu_sc as plsc

_BM = 1024


def _make_gather(F, V, D, B):
    info = plsc.get_sparse_core_info()
    NC, NS = info.num_cores, info.num_subcores
    NW = NC * NS
    assert D == NW
    FD = F * D
    HALF = B // 2
    mesh = plsc.VectorSubcoreMesh(core_axis_name="c", subcore_axis_name="s")

    @functools.partial(
        pl.kernel,
        mesh=mesh,
        out_type=jax.ShapeDtypeStruct((FD, B), jnp.float32),
        scratch_types=[
            pltpu.VMEM((V,), jnp.float32),
            pltpu.VMEM((HALF,), jnp.int32),
            pltpu.VMEM((HALF,), jnp.float32),
        ],
        compiler_params=pltpu.CompilerParams(
            use_tc_tiling_on_sc=True, needs_layout_passes=False),
    )
    def gather(tab_hbm, idx_hbm, out_hbm, row_v, idx_v, out_v):
        w = lax.axis_index("s") * NC + lax.axis_index("c")

        def field(i, carry):
            fd = i * D + w
            pltpu.sync_copy(tab_hbm.at[fd // 8, fd % 8], row_v)

            def half(h, carry2):
                b0 = h * HALF
                pltpu.sync_copy(idx_hbm.at[i, pl.ds(b0, HALF)], idx_v)

                def chunk(c, carry3):
                    for u in range(8):
                        o = (c * 8 + u) * 16
                        iv = idx_v[pl.ds(o, 16)]
                        out_v[pl.ds(o, 16)] = plsc.load_gather(row_v, [iv])
                    return carry3

                lax.fori_loop(0, HALF // 128, chunk, 0)
                pltpu.sync_copy(out_v, out_hbm.at[fd, pl.ds(b0, HALF)])
                return carry2

            lax.fori_loop(0, 2, half, 0)
            return carry

        lax.fori_loop(0, F, field, 0)

    return gather


def _mlp(emb_t, num_t, W1e, W1n, b1, W2, b2, W3, b3, W4, b4):
    FD, Bt = emb_t.shape
    NUM = num_t.shape[0]
    cdim0 = (((0,), (0,)), ((), ()))

    def body(emb_ref, num_ref, w1e_ref, w1n_ref, b1_ref, w2_ref, b2_ref,
             w3_ref, b3_ref, w4_ref, b4_ref, out_ref):
        h = lax.dot_general(emb_ref[...], w1e_ref[...], cdim0,
                            preferred_element_type=jnp.float32)
        h = h + lax.dot_general(num_ref[...], w1n_ref[...], cdim0,
                                preferred_element_type=jnp.float32)
        h = jnp.maximum(h + b1_ref[...], 0.0)
        h = jnp.maximum(jnp.dot(h, w2_ref[...], preferred_element_type=jnp.float32) + b2_ref[...], 0.0)
        h = jnp.maximum(jnp.dot(h, w3_ref[...], preferred_element_type=jnp.float32) + b3_ref[...], 0.0)
        out_ref[...] = lax.dot_general(w4_ref[...], h, (((0,), (1,)), ((), ())),
                                       preferred_element_type=jnp.float32) + b4_ref[...]

    def full(a):
        nd = a.ndim
        return pl.BlockSpec(a.shape, lambda i, _nd=nd: (0,) * _nd)

    return pl.pallas_call(
        body,
        grid=(Bt // _BM,),
        in_specs=[
            pl.BlockSpec((FD, _BM), lambda i: (0, i)),
            pl.BlockSpec((NUM, _BM), lambda i: (0, i)),
            full(W1e), full(W1n), full(b1),
            full(W2), full(b2), full(W3), full(b3), full(W4), full(b4),
        ],
        out_specs=pl.BlockSpec((1, _BM), lambda i: (0, i)),
        out_shape=jax.ShapeDtypeStruct((1, Bt), jnp.float32),
    )(emb_t, num_t, W1e, W1n, b1, W2, b2, W3, b3, W4, b4)


def kernel(categorical_inputs, numeric_inputs, tables, W1, b1, W2, b2, W3, b3, W4, b4):
    B, F = categorical_inputs.shape
    _, V, D = tables.shape
    FD = F * D

    tab_rows = tables.transpose(0, 2, 1).reshape(FD // 8, 8, V)
    idx_t = categorical_inputs.T
    num_t = numeric_inputs.T

    emb_t = _make_gather(F, V, D, B)(tab_rows, idx_t)

    out = _mlp(
        emb_t, num_t,
        W1[:FD], W1[FD:], b1.reshape(1, -1),
        W2, b2.reshape(1, -1), W3, b3.reshape(1, -1), W4, b4.reshape(1, -1),
    )
    return out.reshape(B)
